# baseline (speedup 1.0000x reference)
import functools

import jax
import jax.numpy as jnp
from jax import lax
from jax.experimental import pallas as pl
from jax.experimental.pallas import tpu as pltpu
from jax.experimental.pallas import tpu_sc as plsc

B = 4096
LSEQ = 50
EMB = 256
OUT = 1000
VOCAB = 100000

NC = 2
NS = 16
LANES = 16
NW = NC * NS
NREG = EMB // LANES

NCHUNKS = 2
BC = B // NCHUNKS
BPW = BC // NW
TC_BB = 512
NBLK = BC // TC_BB


def _sc_pool_body(chunk_base, src_hbm, table_hbm, out_hbm, idx_v, rows0,
                  rows1, acc_v, sem0, sem1):
    c = lax.axis_index("c")
    s = lax.axis_index("s")
    wid = s * NC + c
    base = chunk_base + wid * BPW

    pltpu.sync_copy(src_hbm.at[pl.ds(base, BPW), :], idx_v)

    pltpu.async_copy(table_hbm.at[idx_v.at[0]], rows0, sem0)
    pltpu.async_copy(table_hbm.at[idx_v.at[1]], rows1, sem1)

    def process(rows, row_i):
        def jbody(j, accs):
            return tuple(accs[k] + rows[j, pl.ds(k * LANES, LANES)]
                         for k in range(NREG))
        zero = jnp.zeros((LANES,), jnp.float32)
        accs = lax.fori_loop(0, LSEQ, jbody, (zero,) * NREG)
        for k in range(NREG):
            acc_v[row_i, pl.ds(k * LANES, LANES)] = accs[k]

    def obody(i, carry):
        r0 = 2 * i
        pltpu.make_async_copy(table_hbm.at[idx_v.at[r0]], rows0, sem0).wait()
        process(rows0, r0)

        @pl.when(r0 + 2 < BPW)
        def _():
            pltpu.async_copy(table_hbm.at[idx_v.at[r0 + 2]], rows0, sem0)

        pltpu.make_async_copy(table_hbm.at[idx_v.at[r0 + 1]], rows1,
                              sem1).wait()
        process(rows1, r0 + 1)

        @pl.when(r0 + 3 < BPW)
        def _():
            pltpu.async_copy(table_hbm.at[idx_v.at[r0 + 3]], rows1, sem1)

        return carry

    lax.fori_loop(0, BPW // 2, obody, 0)

    pltpu.sync_copy(acc_v, out_hbm.at[pl.ds(wid * BPW, BPW), :])


def _sc_pool(src, table, k):
    mesh = plsc.VectorSubcoreMesh(core_axis_name="c", subcore_axis_name="s")
    f = pl.kernel(
        functools.partial(_sc_pool_body, k * BC),
        out_type=jax.ShapeDtypeStruct((BC, EMB), jnp.float32),
        mesh=mesh,
        scratch_types=[
            pltpu.VMEM((BPW, LSEQ), jnp.int32),
            pltpu.VMEM((LSEQ, EMB), jnp.float32),
            pltpu.VMEM((LSEQ, EMB), jnp.float32),
            pltpu.VMEM((BPW, EMB), jnp.float32),
            pltpu.SemaphoreType.DMA,
            pltpu.SemaphoreType.DMA,
        ],
    )
    return f(src, table)


def _finish_block(emb_blk, src_blk, w, bias):
    cnt = jnp.sum((src_blk != 0).astype(jnp.float32), axis=1, keepdims=True)
    x = emb_blk / cnt
    logits = lax.dot_general(x, w, (((1,), (1,)), ((), ())),
                             preferred_element_type=jnp.float32,
                             precision=lax.Precision.HIGHEST)
    logits = logits + bias
    m = jnp.max(logits, axis=-1, keepdims=True)
    sh = logits - m
    lse = jnp.log(jnp.sum(jnp.exp(sh), axis=-1, keepdims=True))
    return sh - lse


def _tc_finish0_body(emb_ref, src_ref, w_ref, b_ref, out_ref):
    out_ref[...] = _finish_block(emb_ref[...], src_ref[...], w_ref[...],
                                 b_ref[...])


def _tc_finish0(emb0, src, W, b2d):
    return pl.pallas_call(
        _tc_finish0_body,
        grid=(NBLK,),
        in_specs=[
            pl.BlockSpec((TC_BB, EMB), lambda i: (i, 0)),
            pl.BlockSpec((TC_BB, LSEQ), lambda i: (i, 0)),
            pl.BlockSpec((OUT, EMB), lambda i: (0, 0)),
            pl.BlockSpec((1, OUT), lambda i: (0, 0)),
        ],
        out_specs=pl.BlockSpec((TC_BB, OUT), lambda i: (i, 0)),
        out_shape=jax.ShapeDtypeStruct((BC, OUT), jnp.float32),
    )(emb0, src, W, b2d)


def _tc_merge_body(emb_ref, src_ref, w_ref, b_ref, prev_ref, out_ref):
    i = pl.program_id(0)

    @pl.when(i < NBLK)
    def _():
        out_ref[...] = prev_ref[...]

    @pl.when(i >= NBLK)
    def _():
        out_ref[...] = _finish_block(emb_ref[...], src_ref[...], w_ref[...],
                                     b_ref[...])


def _tc_merge(emb1, src, W, b2d, out0):
    return pl.pallas_call(
        _tc_merge_body,
        grid=(2 * NBLK,),
        in_specs=[
            pl.BlockSpec((TC_BB, EMB),
                         lambda i: (jnp.maximum(i - NBLK, 0), 0)),
            pl.BlockSpec((TC_BB, LSEQ),
                         lambda i: (jnp.maximum(i, NBLK), 0)),
            pl.BlockSpec((OUT, EMB), lambda i: (0, 0)),
            pl.BlockSpec((1, OUT), lambda i: (0, 0)),
            pl.BlockSpec((TC_BB, OUT),
                         lambda i: (jnp.minimum(i, NBLK - 1), 0)),
        ],
        out_specs=pl.BlockSpec((TC_BB, OUT), lambda i: (i, 0)),
        out_shape=jax.ShapeDtypeStruct((B, OUT), jnp.float32),
    )(emb1, src, W, b2d, out0)


def kernel(src, table, W, b):
    b2d = b.reshape(1, OUT)
    emb0 = _sc_pool(src, table, 0)
    emb1 = _sc_pool(src, table, 1)
    out0 = _tc_finish0(emb0, src, W, b2d)
    return _tc_merge(emb1, src, W, b2d, out0)


# speedup vs baseline: 1.0285x; 1.0285x over previous
import functools

import jax
import jax.numpy as jnp
from jax import lax
from jax.experimental import pallas as pl
from jax.experimental.pallas import tpu as pltpu
from jax.experimental.pallas import tpu_sc as plsc

B = 4096
LSEQ = 50
EMB = 256
OUT = 1000
VOCAB = 100000

NC = 2
NS = 16
LANES = 16
NW = NC * NS
NREG = EMB // LANES

NCHUNKS = 2
BC = B // NCHUNKS
BPW = BC // NW
TC_BB = 512
NBLK = BC // TC_BB


def _sc_pool_body(chunk_base, src_hbm, table_hbm, out_hbm, idx_v, rows0,
                  rows1, acc_v, sem0, sem1):
    c = lax.axis_index("c")
    s = lax.axis_index("s")
    wid = s * NC + c
    base = chunk_base + wid * BPW

    pltpu.sync_copy(src_hbm.at[pl.ds(base, BPW), :], idx_v)

    pltpu.async_copy(table_hbm.at[idx_v.at[0]], rows0, sem0)
    pltpu.async_copy(table_hbm.at[idx_v.at[1]], rows1, sem1)

    def process(rows, row_i):
        def jbody(j, accs):
            return tuple(accs[k] + rows[j, pl.ds(k * LANES, LANES)]
                         for k in range(NREG))
        zero = jnp.zeros((LANES,), jnp.float32)
        accs = lax.fori_loop(0, LSEQ, jbody, (zero,) * NREG)
        for k in range(NREG):
            acc_v[row_i, pl.ds(k * LANES, LANES)] = accs[k]

    def obody(i, carry):
        r0 = 2 * i
        pltpu.make_async_copy(table_hbm.at[idx_v.at[r0]], rows0, sem0).wait()
        process(rows0, r0)

        @pl.when(r0 + 2 < BPW)
        def _():
            pltpu.async_copy(table_hbm.at[idx_v.at[r0 + 2]], rows0, sem0)

        pltpu.make_async_copy(table_hbm.at[idx_v.at[r0 + 1]], rows1,
                              sem1).wait()
        process(rows1, r0 + 1)

        @pl.when(r0 + 3 < BPW)
        def _():
            pltpu.async_copy(table_hbm.at[idx_v.at[r0 + 3]], rows1, sem1)

        return carry

    lax.fori_loop(0, BPW // 2, obody, 0)

    pltpu.sync_copy(acc_v, out_hbm.at[pl.ds(wid * BPW, BPW), :])


def _sc_pool(src, table, k):
    mesh = plsc.VectorSubcoreMesh(core_axis_name="c", subcore_axis_name="s")
    f = pl.kernel(
        functools.partial(_sc_pool_body, k * BC),
        out_type=jax.ShapeDtypeStruct((BC, EMB), jnp.float32),
        mesh=mesh,
        scratch_types=[
            pltpu.VMEM((BPW, LSEQ), jnp.int32),
            pltpu.VMEM((LSEQ, EMB), jnp.float32),
            pltpu.VMEM((LSEQ, EMB), jnp.float32),
            pltpu.VMEM((BPW, EMB), jnp.float32),
            pltpu.SemaphoreType.DMA,
            pltpu.SemaphoreType.DMA,
        ],
    )
    return f(src, table)


def _finish_block(emb_blk, src_blk, w, bias):
    cnt = jnp.sum((src_blk != 0).astype(jnp.float32), axis=1, keepdims=True)
    x = emb_blk / cnt
    logits = lax.dot_general(x, w, (((1,), (1,)), ((), ())),
                             preferred_element_type=jnp.float32,
                             precision=lax.Precision.HIGHEST)
    logits = logits + bias
    m = jnp.max(logits, axis=-1, keepdims=True)
    sh = logits - m
    lse = jnp.log(jnp.sum(jnp.exp(sh), axis=-1, keepdims=True))
    return sh - lse


def _tc_finishk_body(emb_ref, src_ref, w_ref, b_ref, prev_ref, out_ref):
    del prev_ref
    out_ref[...] = _finish_block(emb_ref[...], src_ref[...], w_ref[...],
                                 b_ref[...])


def _tc_finishk(emb_k, src, W, b2d, k, prev):
    blk0 = k * NBLK
    return pl.pallas_call(
        _tc_finishk_body,
        grid=(NBLK,),
        in_specs=[
            pl.BlockSpec((TC_BB, EMB), lambda i: (i, 0)),
            pl.BlockSpec((TC_BB, LSEQ), lambda i, b0=blk0: (i + b0, 0)),
            pl.BlockSpec((OUT, EMB), lambda i: (0, 0)),
            pl.BlockSpec((1, OUT), lambda i: (0, 0)),
            pl.BlockSpec(memory_space=pl.ANY),
        ],
        out_specs=pl.BlockSpec((TC_BB, OUT), lambda i, b0=blk0: (i + b0, 0)),
        out_shape=jax.ShapeDtypeStruct((B, OUT), jnp.float32),
        input_output_aliases={4: 0},
    )(emb_k, src, W, b2d, prev)


def _tc_first_body(emb_ref, src_ref, w_ref, b_ref, out_ref):
    out_ref[...] = _finish_block(emb_ref[...], src_ref[...], w_ref[...],
                                 b_ref[...])


def _tc_first(emb0, src, W, b2d):
    return pl.pallas_call(
        _tc_first_body,
        grid=(NBLK,),
        in_specs=[
            pl.BlockSpec((TC_BB, EMB), lambda i: (i, 0)),
            pl.BlockSpec((TC_BB, LSEQ), lambda i: (i, 0)),
            pl.BlockSpec((OUT, EMB), lambda i: (0, 0)),
            pl.BlockSpec((1, OUT), lambda i: (0, 0)),
        ],
        out_specs=pl.BlockSpec((TC_BB, OUT), lambda i: (i, 0)),
        out_shape=jax.ShapeDtypeStruct((B, OUT), jnp.float32),
    )(emb0, src, W, b2d)


def kernel(src, table, W, b):
    b2d = b.reshape(1, OUT)
    out = None
    gate_src, gate_table = src, table
    for k in range(NCHUNKS):
        emb_k = _sc_pool(gate_src, gate_table, k)
        if k + 1 < NCHUNKS:
            emb_k, gate_src, gate_table = lax.optimization_barrier(
                (emb_k, src, table))
        if k == 0:
            out = _tc_first(emb_k, src, W, b2d)
        else:
            out = _tc_finishk(emb_k, src, W, b2d, k, out)
    return out
